# baseline (speedup 1.0000x reference)
import functools

import jax
from jax import lax
from jax.experimental import pallas as pl
from jax.experimental.pallas import tpu as pltpu
from jax.experimental.pallas import tpu_sc as plsc


def kernel(x, pe):
    seq_len = x.shape[1]
    d_model = pe.shape[2]
    info = plsc.get_sparse_core_info()
    nw = info.num_cores * info.num_subcores
    rows_per_w = seq_len // nw
    mesh = plsc.VectorSubcoreMesh(core_axis_name="c", subcore_axis_name="s")

    @functools.partial(
        pl.kernel,
        mesh=mesh,
        out_type=jax.ShapeDtypeStruct((1, seq_len, d_model), pe.dtype),
        scratch_types=[pltpu.SemaphoreType.DMA],
    )
    def sc_copy(pe_hbm, out_hbm, sem):
        wid = lax.axis_index("s") * info.num_cores + lax.axis_index("c")
        base = wid * rows_per_w
        pltpu.async_copy(
            pe_hbm.at[:, pl.ds(base, rows_per_w), :],
            out_hbm.at[:, pl.ds(base, rows_per_w), :],
            sem,
        ).wait()

    return sc_copy(pe)


# speedup vs baseline: 10.9296x; 10.9296x over previous
import functools

import jax
from jax import lax
from jax.experimental import pallas as pl
from jax.experimental.pallas import tpu as pltpu
from jax.experimental.pallas import tpu_sc as plsc

_N_CHUNKS = 2


def kernel(x, pe):
    seq_len = x.shape[1]
    d_model = pe.shape[2]
    info = plsc.get_sparse_core_info()
    nw = info.num_cores * info.num_subcores
    rows_per_w = seq_len // nw
    chunk_rows = rows_per_w // _N_CHUNKS
    mesh = plsc.VectorSubcoreMesh(core_axis_name="c", subcore_axis_name="s")

    @functools.partial(
        pl.kernel,
        mesh=mesh,
        out_type=jax.ShapeDtypeStruct((1, seq_len, d_model), pe.dtype),
        scratch_types=[
            pltpu.VMEM((_N_CHUNKS, chunk_rows, d_model), pe.dtype),
            pltpu.SemaphoreType.DMA((_N_CHUNKS,)),
            pltpu.SemaphoreType.DMA((_N_CHUNKS,)),
        ],
    )
    def sc_copy(pe_hbm, out_hbm, buf, in_sems, out_sems):
        wid = lax.axis_index("s") * info.num_cores + lax.axis_index("c")
        base = wid * rows_per_w

        def cp_in(i):
            return pltpu.make_async_copy(
                pe_hbm.at[0, pl.ds(base + i * chunk_rows, chunk_rows), :],
                buf.at[i],
                in_sems.at[i],
            )

        def cp_out(i):
            return pltpu.make_async_copy(
                buf.at[i],
                out_hbm.at[0, pl.ds(base + i * chunk_rows, chunk_rows), :],
                out_sems.at[i],
            )

        for i in range(_N_CHUNKS):
            cp_in(i).start()
        for i in range(_N_CHUNKS):
            cp_in(i).wait()
            cp_out(i).start()
        for i in range(_N_CHUNKS):
            cp_out(i).wait()

    return sc_copy(pe)


# speedup vs baseline: 26.7436x; 2.4469x over previous
import jax
from jax.experimental import pallas as pl
from jax.experimental.pallas import tpu as pltpu

_CHUNK_ROWS = 512


def _copy_body(pe_ref, out_ref, buf, in_sems, out_sems):
    seq_len = out_ref.shape[1]
    n_chunks = seq_len // _CHUNK_ROWS

    def cp_in(i):
        return pltpu.make_async_copy(
            pe_ref.at[:, pl.ds(i * _CHUNK_ROWS, _CHUNK_ROWS), :],
            buf.at[i],
            in_sems.at[i],
        )

    def cp_out(i):
        return pltpu.make_async_copy(
            buf.at[i],
            out_ref.at[:, pl.ds(i * _CHUNK_ROWS, _CHUNK_ROWS), :],
            out_sems.at[i],
        )

    cp_in(0).start()
    for i in range(n_chunks):
        cp_in(i).wait()
        if i + 1 < n_chunks:
            cp_in(i + 1).start()
        cp_out(i).start()
    for i in range(n_chunks):
        cp_out(i).wait()


def kernel(x, pe):
    seq_len = x.shape[1]
    d_model = pe.shape[2]
    n_chunks = seq_len // _CHUNK_ROWS
    out_shape = jax.ShapeDtypeStruct((1, seq_len, d_model), pe.dtype)
    return pl.pallas_call(
        _copy_body,
        out_shape=out_shape,
        in_specs=[pl.BlockSpec(memory_space=pl.ANY)],
        out_specs=pl.BlockSpec(memory_space=pl.ANY),
        scratch_shapes=[
            pltpu.VMEM((n_chunks, 1, _CHUNK_ROWS, d_model), pe.dtype),
            pltpu.SemaphoreType.DMA((n_chunks,)),
            pltpu.SemaphoreType.DMA((n_chunks,)),
        ],
    )(pe)


# speedup vs baseline: 46.1899x; 1.7271x over previous
import jax
from jax.experimental import pallas as pl
from jax.experimental.pallas import tpu as pltpu

_CHUNK_ROWS = 1024


def _copy_body(pe_ref, out_ref, buf, in_sems, out_sems):
    seq_len = out_ref.shape[1]
    n_chunks = seq_len // _CHUNK_ROWS

    def cp_in(i):
        return pltpu.make_async_copy(
            pe_ref.at[:, pl.ds(i * _CHUNK_ROWS, _CHUNK_ROWS), :],
            buf.at[i],
            in_sems.at[i],
        )

    def cp_out(i):
        return pltpu.make_async_copy(
            buf.at[i],
            out_ref.at[:, pl.ds(i * _CHUNK_ROWS, _CHUNK_ROWS), :],
            out_sems.at[i],
        )

    for i in range(n_chunks):
        cp_in(i).start()
    for i in range(n_chunks):
        cp_in(i).wait()
        cp_out(i).start()
    for i in range(n_chunks):
        cp_out(i).wait()


def kernel(x, pe):
    seq_len = x.shape[1]
    d_model = pe.shape[2]
    n_chunks = seq_len // _CHUNK_ROWS
    out_shape = jax.ShapeDtypeStruct((1, seq_len, d_model), pe.dtype)
    return pl.pallas_call(
        _copy_body,
        out_shape=out_shape,
        in_specs=[pl.BlockSpec(memory_space=pl.ANY)],
        out_specs=pl.BlockSpec(memory_space=pl.ANY),
        scratch_shapes=[
            pltpu.VMEM((n_chunks, 1, _CHUNK_ROWS, d_model), pe.dtype),
            pltpu.SemaphoreType.DMA((n_chunks,)),
            pltpu.SemaphoreType.DMA((n_chunks,)),
        ],
    )(pe)
